# baseline (speedup 1.0000x reference)
import functools

import jax
import jax.numpy as jnp
from jax import lax
from jax.experimental import pallas as pl
from jax.experimental.pallas import tpu as pltpu
from jax.experimental.pallas import tpu_sc as plsc

NUM_SEGS = 64
ROWS = 100000
EPS = 1e-8

CH = 200
NCH = ROWS // CH
NW = 32
KMAX = -(-NCH // NW)

BLK = 20000
NB = ROWS // BLK


def _sc_stats_body(x_hbm, b_hbm, sums_hbm, sqs_hbm, cnts_hbm,
                   x_t, sq_t, one_t, zero_t, idx_t,
                   sum_sh, sq_sh, cnt_sh):
    cid = lax.axis_index("c")
    sid = lax.axis_index("s")
    wid = sid * 2 + cid

    def fill_zero(r, carry):
        for j in range(8):
            zero_t[r, pl.ds(j * 16, 16)] = jnp.zeros((16,), jnp.float32)
        return carry

    lax.fori_loop(0, NUM_SEGS, fill_zero, 0)

    def fill_one(r, carry):
        for j in range(8):
            one_t[r, pl.ds(j * 16, 16)] = jnp.full((16,), 1.0, jnp.float32)
        return carry

    lax.fori_loop(0, CH, fill_one, 0)

    @pl.when(sid == 0)
    def _init():
        pltpu.sync_copy(zero_t, sum_sh)
        pltpu.sync_copy(zero_t, sq_sh)
        pltpu.sync_copy(zero_t, cnt_sh)

    plsc.subcore_barrier()

    for k in range(KMAX):
        c = wid + k * NW

        @pl.when(c < NCH)
        def _chunk():
            base = c * CH
            pltpu.sync_copy(x_hbm.at[pl.ds(base, CH)], x_t)
            pltpu.sync_copy(b_hbm.at[pl.ds(base, CH)], idx_t)

            def square_row(r, carry):
                for j in range(8):
                    v = x_t[r, pl.ds(j * 16, 16)]
                    sq_t[r, pl.ds(j * 16, 16)] = v * v
                return carry

            lax.fori_loop(0, CH, square_row, 0)
            pltpu.sync_copy(x_t, sum_sh.at[idx_t], add=True)
            pltpu.sync_copy(sq_t, sq_sh.at[idx_t], add=True)
            pltpu.sync_copy(one_t, cnt_sh.at[idx_t], add=True)

    plsc.subcore_barrier()

    @pl.when(sid == 0)
    def _flush():
        pltpu.sync_copy(sum_sh, sums_hbm.at[cid])
        pltpu.sync_copy(sq_sh, sqs_hbm.at[cid])
        pltpu.sync_copy(cnt_sh, cnts_hbm.at[cid])


def _sc_stats(x, batch_i32):
    mesh = plsc.VectorSubcoreMesh(core_axis_name="c", subcore_axis_name="s")
    part = jax.ShapeDtypeStruct((2, NUM_SEGS, 128), jnp.float32)
    fn = functools.partial(
        pl.kernel,
        mesh=mesh,
        out_type=[part, part, part],
        scratch_types=[
            pltpu.VMEM((CH, 128), jnp.float32),
            pltpu.VMEM((CH, 128), jnp.float32),
            pltpu.VMEM((CH, 128), jnp.float32),
            pltpu.VMEM((NUM_SEGS, 128), jnp.float32),
            pltpu.VMEM((CH,), jnp.int32),
            pltpu.VMEM_SHARED((NUM_SEGS, 128), jnp.float32),
            pltpu.VMEM_SHARED((NUM_SEGS, 128), jnp.float32),
            pltpu.VMEM_SHARED((NUM_SEGS, 128), jnp.float32),
        ],
    )(_sc_stats_body)
    return fn(x, batch_i32)


def _norm_body(batch_ref, x_ref, sums_ref, sqs_ref, cnts_ref, w_ref, bia_ref,
               ms_ref, out_ref):
    cnt = jnp.maximum(cnts_ref[0] + cnts_ref[1], 1.0)
    seg_sum = sums_ref[0] + sums_ref[1]
    seg_sq = sqs_ref[0] + sqs_ref[1]
    mean = seg_sum / cnt
    var = (seg_sq - cnt * mean * mean) / jnp.maximum(cnt - 1.0, 1.0)
    std = jnp.sqrt(jnp.maximum(var, 0.0)) + EPS
    a = w_ref[...] / std
    bcoef = bia_ref[...] - a * mean * ms_ref[...]

    b = batch_ref[0]
    seg_ids = lax.broadcasted_iota(jnp.int32, (BLK, NUM_SEGS), 1)
    oh = (jnp.broadcast_to(b.reshape(BLK, 1), (BLK, NUM_SEGS)) == seg_ids
          ).astype(jnp.bfloat16)
    ab = jnp.concatenate([a, bcoef], axis=1).astype(jnp.bfloat16)
    dn = (((1,), (0,)), ((), ()))
    ab_rows = lax.dot_general(oh, ab, dn, preferred_element_type=jnp.float32)
    out_ref[...] = x_ref[...] * ab_rows[:, :128] + ab_rows[:, 128:]


@functools.partial(jax.jit, static_argnames=("interpret",))
def kernel(x, batch, weight, bias, mean_scale, interpret=False):
    batch_i32 = batch.astype(jnp.int32)
    sums, sqs, cnts = _sc_stats(x, batch_i32)

    batch3 = batch_i32.reshape(NB, 1, BLK)
    out = pl.pallas_call(
        _norm_body,
        grid=(NB,),
        in_specs=[
            pl.BlockSpec((1, 1, BLK), lambda i: (i, 0, 0)),
            pl.BlockSpec((BLK, 128), lambda i: (i, 0)),
            pl.BlockSpec((2, NUM_SEGS, 128), lambda i: (0, 0, 0)),
            pl.BlockSpec((2, NUM_SEGS, 128), lambda i: (0, 0, 0)),
            pl.BlockSpec((2, NUM_SEGS, 128), lambda i: (0, 0, 0)),
            pl.BlockSpec((1, 128), lambda i: (0, 0)),
            pl.BlockSpec((1, 128), lambda i: (0, 0)),
            pl.BlockSpec((1, 128), lambda i: (0, 0)),
        ],
        out_specs=pl.BlockSpec((BLK, 128), lambda i: (i, 0)),
        out_shape=jax.ShapeDtypeStruct((ROWS, 128), jnp.float32),
        interpret=interpret,
    )(batch3, x, sums, sqs, cnts, weight.reshape(1, 128), bias.reshape(1, 128),
      mean_scale.reshape(1, 128))
    return out


# speedup vs baseline: 1.1281x; 1.1281x over previous
import functools

import jax
import jax.numpy as jnp
from jax import lax
from jax.experimental import pallas as pl
from jax.experimental.pallas import tpu as pltpu
from jax.experimental.pallas import tpu_sc as plsc

NUM_SEGS = 64
ROWS = 100000
EPS = 1e-8

CH = 200
NCH = ROWS // CH
NW = 32
KMAX = -(-NCH // NW)

BLK = 20000
NB = ROWS // BLK


def _sc_stats_body(x_hbm, b_hbm, sums_hbm, sqs_hbm,
                   x_t, sq_t, zero_t, idx_t,
                   sum_sh, sq_sh):
    cid = lax.axis_index("c")
    sid = lax.axis_index("s")
    wid = sid * 2 + cid

    def fill_zero(r, carry):
        for j in range(8):
            zero_t[r, pl.ds(j * 16, 16)] = jnp.zeros((16,), jnp.float32)
        return carry

    lax.fori_loop(0, NUM_SEGS, fill_zero, 0)

    @pl.when(sid == 0)
    def _init():
        pltpu.sync_copy(zero_t, sum_sh)
        pltpu.sync_copy(zero_t, sq_sh)

    plsc.subcore_barrier()

    for k in range(KMAX):
        c = wid + k * NW

        @pl.when(c < NCH)
        def _chunk():
            base = c * CH
            pltpu.sync_copy(x_hbm.at[pl.ds(base, CH)], x_t)
            pltpu.sync_copy(b_hbm.at[pl.ds(base, CH)], idx_t)

            def square_row(r, carry):
                for j in range(8):
                    v = x_t[r, pl.ds(j * 16, 16)]
                    sq_t[r, pl.ds(j * 16, 16)] = v * v
                return carry

            lax.fori_loop(0, CH, square_row, 0)
            pltpu.sync_copy(x_t, sum_sh.at[idx_t], add=True)
            pltpu.sync_copy(sq_t, sq_sh.at[idx_t], add=True)

    plsc.subcore_barrier()

    @pl.when(sid == 0)
    def _flush():
        pltpu.sync_copy(sum_sh, sums_hbm.at[cid])
        pltpu.sync_copy(sq_sh, sqs_hbm.at[cid])


def _sc_stats(x, batch_i32):
    mesh = plsc.VectorSubcoreMesh(core_axis_name="c", subcore_axis_name="s")
    part = jax.ShapeDtypeStruct((2, NUM_SEGS, 128), jnp.float32)
    fn = functools.partial(
        pl.kernel,
        mesh=mesh,
        out_type=[part, part],
        scratch_types=[
            pltpu.VMEM((CH, 128), jnp.float32),
            pltpu.VMEM((CH, 128), jnp.float32),
            pltpu.VMEM((NUM_SEGS, 128), jnp.float32),
            pltpu.VMEM((CH,), jnp.int32),
            pltpu.VMEM_SHARED((NUM_SEGS, 128), jnp.float32),
            pltpu.VMEM_SHARED((NUM_SEGS, 128), jnp.float32),
        ],
    )(_sc_stats_body)
    return fn(x, batch_i32)


def _count_body(batch_ref, out_ref):
    i = pl.program_id(0)

    @pl.when(i == 0)
    def _zero():
        out_ref[...] = jnp.zeros((NUM_SEGS, 128), jnp.float32)

    b = batch_ref[0]
    seg_ids = lax.broadcasted_iota(jnp.int32, (NUM_SEGS, BLK), 0)
    oh = (jnp.broadcast_to(b.reshape(1, BLK), (NUM_SEGS, BLK)) == seg_ids
          ).astype(jnp.bfloat16)
    ones = jnp.ones((BLK, 128), jnp.bfloat16)
    dn = (((1,), (0,)), ((), ()))
    out_ref[...] += lax.dot_general(oh, ones, dn,
                                    preferred_element_type=jnp.float32)


def _norm_body(batch_ref, x_ref, sums_ref, sqs_ref, cnts_ref, w_ref, bia_ref,
               ms_ref, out_ref):
    cnt = jnp.maximum(cnts_ref[...], 1.0)
    seg_sum = sums_ref[0] + sums_ref[1]
    seg_sq = sqs_ref[0] + sqs_ref[1]
    mean = seg_sum / cnt
    var = (seg_sq - cnt * mean * mean) / jnp.maximum(cnt - 1.0, 1.0)
    std = jnp.sqrt(jnp.maximum(var, 0.0)) + EPS
    a = w_ref[...] / std
    bcoef = bia_ref[...] - a * mean * ms_ref[...]

    b = batch_ref[0]
    seg_ids = lax.broadcasted_iota(jnp.int32, (BLK, NUM_SEGS), 1)
    oh = (jnp.broadcast_to(b.reshape(BLK, 1), (BLK, NUM_SEGS)) == seg_ids
          ).astype(jnp.bfloat16)
    ab = jnp.concatenate([a, bcoef], axis=1).astype(jnp.bfloat16)
    dn = (((1,), (0,)), ((), ()))
    ab_rows = lax.dot_general(oh, ab, dn, preferred_element_type=jnp.float32)
    out_ref[...] = x_ref[...] * ab_rows[:, :128] + ab_rows[:, 128:]


@functools.partial(jax.jit, static_argnames=("interpret",))
def kernel(x, batch, weight, bias, mean_scale, interpret=False):
    batch_i32 = batch.astype(jnp.int32)
    sums, sqs = _sc_stats(x, batch_i32)

    batch3 = batch_i32.reshape(NB, 1, BLK)
    cnts = pl.pallas_call(
        _count_body,
        grid=(NB,),
        in_specs=[pl.BlockSpec((1, 1, BLK), lambda i: (i, 0, 0))],
        out_specs=pl.BlockSpec((NUM_SEGS, 128), lambda i: (0, 0)),
        out_shape=jax.ShapeDtypeStruct((NUM_SEGS, 128), jnp.float32),
        interpret=interpret,
    )(batch3)

    out = pl.pallas_call(
        _norm_body,
        grid=(NB,),
        in_specs=[
            pl.BlockSpec((1, 1, BLK), lambda i: (i, 0, 0)),
            pl.BlockSpec((BLK, 128), lambda i: (i, 0)),
            pl.BlockSpec((2, NUM_SEGS, 128), lambda i: (0, 0, 0)),
            pl.BlockSpec((2, NUM_SEGS, 128), lambda i: (0, 0, 0)),
            pl.BlockSpec((NUM_SEGS, 128), lambda i: (0, 0)),
            pl.BlockSpec((1, 128), lambda i: (0, 0)),
            pl.BlockSpec((1, 128), lambda i: (0, 0)),
            pl.BlockSpec((1, 128), lambda i: (0, 0)),
        ],
        out_specs=pl.BlockSpec((BLK, 128), lambda i: (i, 0)),
        out_shape=jax.ShapeDtypeStruct((ROWS, 128), jnp.float32),
        interpret=interpret,
    )(batch3, x, sums, sqs, cnts, weight.reshape(1, 128), bias.reshape(1, 128),
      mean_scale.reshape(1, 128))
    return out
